# baseline (speedup 1.0000x reference)
import functools
import math

import jax
import jax.numpy as jnp
from jax import lax
from jax.experimental import pallas as pl
from jax.experimental.pallas import tpu as pltpu
from jax.experimental.pallas import tpu_sc as plsc

D_MODEL = 64
SCALE = math.sqrt(D_MODEL)
NC, NS, LANES = 2, 16, 16
NW = NC * NS

CHUNK = 128


def _sc_embed(xf, lut):
    B = xf.shape[0]
    b_per_w = B // NW
    n_chunks = b_per_w // CHUNK
    mesh = plsc.VectorSubcoreMesh(core_axis_name="c", subcore_axis_name="s")

    @functools.partial(
        pl.kernel,
        out_type=jax.ShapeDtypeStruct((B, D_MODEL), jnp.float32),
        mesh=mesh,
        compiler_params=pltpu.CompilerParams(use_tc_tiling_on_sc=False),
        scratch_types=[
            pltpu.VMEM((CHUNK,), jnp.int32),
            pltpu.VMEM((CHUNK, D_MODEL), jnp.float32),
            pltpu.SemaphoreType.DMA,
        ],
    )
    def k(x_hbm, lut_hbm, out_hbm, idx_v, rows_v, gsem):
        wid = lax.axis_index("s") * NC + lax.axis_index("c")
        base = wid * b_per_w

        @pl.loop(0, n_chunks)
        def _(g):
            off = base + g * CHUNK
            pltpu.sync_copy(x_hbm.at[pl.ds(off, CHUNK)], idx_v)
            pltpu.async_copy(lut_hbm.at[idx_v], rows_v, gsem).wait()

            @pl.loop(0, CHUNK)
            def _(i):
                for j in range(D_MODEL // LANES):
                    sl = (i, pl.ds(j * LANES, LANES))
                    rows_v[sl] = rows_v[sl] * SCALE

            pltpu.sync_copy(rows_v, out_hbm.at[pl.ds(off, CHUNK)])

    return k(xf, lut)


def kernel(x, lut):
    B0, S = x.shape
    out = _sc_embed(x.reshape(B0 * S), lut)
    return out.reshape(B0, S, D_MODEL)


# speedup vs baseline: 1.2664x; 1.2664x over previous
import functools
import math

import jax
import jax.numpy as jnp
from jax import lax
from jax.experimental import pallas as pl
from jax.experimental.pallas import tpu as pltpu
from jax.experimental.pallas import tpu_sc as plsc

D_MODEL = 64
SCALE = math.sqrt(D_MODEL)
NC, NS, LANES = 2, 16, 16
NW = NC * NS

CHUNK = 128
NBUF = 4


def _sc_embed(xf, lut):
    B = xf.shape[0]
    b_per_w = B // NW
    n_chunks = b_per_w // CHUNK
    n_outer = n_chunks // NBUF
    mesh = plsc.VectorSubcoreMesh(core_axis_name="c", subcore_axis_name="s")

    @functools.partial(
        pl.kernel,
        out_type=jax.ShapeDtypeStruct((B, D_MODEL), jnp.float32),
        mesh=mesh,
        compiler_params=pltpu.CompilerParams(use_tc_tiling_on_sc=False),
        scratch_types=[
            pltpu.VMEM((b_per_w,), jnp.int32),
            pltpu.VMEM((NBUF, CHUNK, D_MODEL), jnp.float32),
            [pltpu.SemaphoreType.DMA] * NBUF,
            [pltpu.SemaphoreType.DMA] * NBUF,
        ],
    )
    def k(x_hbm, lut_hbm, out_hbm, idx_v, rows_v, gsems, wsems):
        wid = lax.axis_index("s") * NC + lax.axis_index("c")
        base = wid * b_per_w
        pltpu.sync_copy(x_hbm.at[pl.ds(base, b_per_w)], idx_v)

        def gather_desc(g, s):
            return pltpu.make_async_copy(
                lut_hbm.at[idx_v.at[pl.ds(g * CHUNK, CHUNK)]],
                rows_v.at[s],
                gsems[s],
            )

        def write_desc(g, s):
            return pltpu.make_async_copy(
                rows_v.at[s],
                out_hbm.at[pl.ds(base + g * CHUNK, CHUNK)],
                wsems[s],
            )

        gather_desc(0, 0).start()
        gather_desc(1, 1).start()

        @pl.loop(0, n_outer)
        def _(t):
            g0 = t * NBUF
            for b in range(NBUF):
                g = g0 + b
                pn = (b + 2) % NBUF

                @pl.when(g + 2 < n_chunks)
                def _():
                    @pl.when(g >= 2)
                    def _():
                        write_desc(g - 2, pn).wait()

                    gather_desc(g + 2, pn).start()

                gather_desc(g, b).wait()

                row_ref = rows_v.at[b]

                @plsc.parallel_loop(0, CHUNK)
                def _(i):
                    for j in range(D_MODEL // LANES):
                        sl = (i, pl.ds(j * LANES, LANES))
                        row_ref[sl] = row_ref[sl] * SCALE

                write_desc(g, b).start()

        write_desc(n_chunks - 2, (n_chunks - 2) % NBUF).wait()
        write_desc(n_chunks - 1, (n_chunks - 1) % NBUF).wait()

    return k(xf, lut)


def kernel(x, lut):
    B0, S = x.shape
    out = _sc_embed(x.reshape(B0 * S), lut)
    return out.reshape(B0, S, D_MODEL)
